# Initial kernel scaffold; baseline (speedup 1.0000x reference)
import jax
import jax.numpy as jnp
from jax.experimental import pallas as pl
from jax.experimental.pallas import tpu as pltpu

N = 16384
D = 64
K = 512
BLK = 2048
GRID = N // BLK


def _body(z_ref, c_ref, w_ref, out_ref, sums_ref, counts_ref, dc_ref):
    i = pl.program_id(0)

    @pl.when(i == 0)
    def _init():
        sums_ref[...] = jnp.zeros_like(sums_ref)
        counts_ref[...] = jnp.zeros_like(counts_ref)
        dc_ref[0, 0] = 0.0

    z = z_ref[...]
    c = c_ref[...]
    zz = jnp.sum(z * z, axis=1, keepdims=True)
    cc = jnp.sum(c * c, axis=1).reshape(1, K)
    zc = jax.lax.dot_general(
        z, c, dimension_numbers=(((1,), (1,)), ((), ())),
        preferred_element_type=jnp.float32,
        precision=jax.lax.Precision.HIGHEST,
    )
    d2 = jnp.maximum(zz - 2.0 * zc + cc, 0.0)

    m = jnp.min(d2, axis=1, keepdims=True)
    iota = jax.lax.broadcasted_iota(jnp.int32, (BLK, K), 1)
    masked_iota = jnp.where(d2 == m, iota, K)
    amin = jnp.min(masked_iota, axis=1, keepdims=True)
    oh = (iota == amin).astype(jnp.float32)

    sums_ref[...] += jax.lax.dot_general(
        oh, z, dimension_numbers=(((0,), (0,)), ((), ())),
        preferred_element_type=jnp.float32,
        precision=jax.lax.Precision.HIGHEST,
    )
    counts_ref[...] += jnp.sum(oh, axis=0, keepdims=True)
    dc_ref[0, 0] += jnp.sum(jnp.sqrt(m + 1e-12))

    @pl.when(i == GRID - 1)
    def _finish():
        counts = counts_ref[0, :]
        means = sums_ref[...] / jnp.maximum(counts, 1.0)[:, None]
        diff = c_ref[...] - means
        cd = jnp.sqrt(jnp.sum(diff * diff, axis=1) + 1e-12)
        nonempty = (counts > 0.0).astype(jnp.float32)
        w = w_ref[0, :]
        nw = nonempty * w
        nc = jnp.sum(nw * cd) / jnp.maximum(jnp.sum(nw), 1e-12)
        out_ref[0, 0] = nc + dc_ref[0, 0] / N


@jax.jit
def kernel(z, centers, weights):
    out = pl.pallas_call(
        _body,
        grid=(GRID,),
        in_specs=[
            pl.BlockSpec((BLK, D), lambda i: (i, 0)),
            pl.BlockSpec((K, D), lambda i: (0, 0)),
            pl.BlockSpec((1, K), lambda i: (0, 0)),
        ],
        out_specs=pl.BlockSpec((1, 1), lambda i: (0, 0)),
        out_shape=jax.ShapeDtypeStruct((1, 1), jnp.float32),
        scratch_shapes=[
            pltpu.VMEM((K, D), jnp.float32),
            pltpu.VMEM((1, K), jnp.float32),
            pltpu.SMEM((1, 1), jnp.float32),
        ],
    )(z, centers, weights.reshape(1, K))
    return out[0, 0]


# speedup vs baseline: 2.3182x; 2.3182x over previous
import jax
import jax.numpy as jnp
from jax.experimental import pallas as pl
from jax.experimental.pallas import tpu as pltpu

N = 16384
D = 64
K = 512
BLK = 2048
GRID = N // BLK


def _body(z_ref, c_ref, w_ref, out_ref, sums_ref, counts_ref, dc_ref):
    i = pl.program_id(0)

    @pl.when(i == 0)
    def _init():
        sums_ref[...] = jnp.zeros_like(sums_ref)
        counts_ref[...] = jnp.zeros_like(counts_ref)
        dc_ref[0, 0] = 0.0

    z = z_ref[...]
    c = c_ref[...]
    zz = jnp.sum(z * z, axis=1, keepdims=True)
    cc = jnp.sum(c * c, axis=1).reshape(1, K)
    zc = jax.lax.dot_general(
        z, c, dimension_numbers=(((1,), (1,)), ((), ())),
        preferred_element_type=jnp.float32,
        precision=jax.lax.Precision.HIGHEST,
    )
    d2 = jnp.maximum(zz - 2.0 * zc + cc, 0.0)

    m = jnp.min(d2, axis=1, keepdims=True)
    iota = jax.lax.broadcasted_iota(jnp.int32, (BLK, K), 1)
    masked_iota = jnp.where(d2 == m, iota, K)
    amin = jnp.min(masked_iota, axis=1, keepdims=True)
    oh = (iota == amin).astype(jnp.float32)

    sums_ref[...] += jax.lax.dot_general(
        oh, z, dimension_numbers=(((0,), (0,)), ((), ())),
        preferred_element_type=jnp.float32,
        precision=jax.lax.Precision.HIGHEST,
    )
    counts_ref[...] += jnp.sum(oh, axis=0, keepdims=True)
    dc_ref[0, 0] += jnp.sum(jnp.sqrt(m + 1e-12))

    @pl.when(i == GRID - 1)
    def _finish():
        counts = counts_ref[0, :]
        means = sums_ref[...] / jnp.maximum(counts, 1.0)[:, None]
        diff = c_ref[...] - means
        cd = jnp.sqrt(jnp.sum(diff * diff, axis=1) + 1e-12)
        nonempty = (counts > 0.0).astype(jnp.float32)
        w = w_ref[0, :]
        nw = nonempty * w
        nc = jnp.sum(nw * cd) / jnp.maximum(jnp.sum(nw), 1e-12)
        out_ref[...] = jnp.full((1, 1), nc + dc_ref[0, 0] / N, jnp.float32)


@jax.jit
def kernel(z, centers, weights):
    out = pl.pallas_call(
        _body,
        grid=(GRID,),
        in_specs=[
            pl.BlockSpec((BLK, D), lambda i: (i, 0)),
            pl.BlockSpec((K, D), lambda i: (0, 0)),
            pl.BlockSpec((1, K), lambda i: (0, 0)),
        ],
        out_specs=pl.BlockSpec((1, 1), lambda i: (0, 0)),
        out_shape=jax.ShapeDtypeStruct((1, 1), jnp.float32),
        scratch_shapes=[
            pltpu.VMEM((K, D), jnp.float32),
            pltpu.VMEM((1, K), jnp.float32),
            pltpu.SMEM((1, 1), jnp.float32),
        ],
    )(z, centers, weights.reshape(1, K))
    return out[0, 0]


# speedup vs baseline: 4.6245x; 1.9949x over previous
import jax
import jax.numpy as jnp
from jax.experimental import pallas as pl
from jax.experimental.pallas import tpu as pltpu

N = 16384
D = 64
K = 512
BLK = 2048
GRID = N // BLK


def _body(z_ref, c_ref, w_ref, out_ref, sums_ref, counts_ref, dc_ref):
    i = pl.program_id(0)

    @pl.when(i == 0)
    def _init():
        sums_ref[...] = jnp.zeros_like(sums_ref)
        counts_ref[...] = jnp.zeros_like(counts_ref)
        dc_ref[0, 0] = 0.0

    z = z_ref[...]
    c = c_ref[...]
    zz = jnp.sum(z * z, axis=1, keepdims=True)
    cc = jnp.sum(c * c, axis=1).reshape(1, K)
    zc = jax.lax.dot_general(
        z, c, dimension_numbers=(((1,), (1,)), ((), ())),
        preferred_element_type=jnp.float32,
    )
    d2 = jnp.maximum(zz - 2.0 * zc + cc, 0.0)
    m = jnp.min(d2, axis=1, keepdims=True)
    iota = jax.lax.broadcasted_iota(jnp.int32, (BLK, K), 1)
    masked_iota = jnp.where(d2 == m, iota, K)
    amin = jnp.min(masked_iota, axis=1, keepdims=True)
    oh = (iota == amin).astype(jnp.float32)

    sums_ref[...] += jax.lax.dot_general(
        oh, z, dimension_numbers=(((0,), (0,)), ((), ())),
        preferred_element_type=jnp.float32,
    )
    counts_ref[...] += jnp.sum(oh, axis=0, keepdims=True)
    dc_ref[0, 0] += jnp.sum(jnp.sqrt(m + 1e-12))

    @pl.when(i == GRID - 1)
    def _finish():
        counts = counts_ref[0, :]
        means = sums_ref[...] / jnp.maximum(counts, 1.0)[:, None]
        diff = c_ref[...] - means
        cd = jnp.sqrt(jnp.sum(diff * diff, axis=1) + 1e-12)
        nonempty = (counts > 0.0).astype(jnp.float32)
        w = w_ref[0, :]
        nw = nonempty * w
        nc = jnp.sum(nw * cd) / jnp.maximum(jnp.sum(nw), 1e-12)
        out_ref[...] = jnp.full((1, 1), nc + dc_ref[0, 0] / N, jnp.float32)


@jax.jit
def kernel(z, centers, weights):
    out = pl.pallas_call(
        _body,
        grid=(GRID,),
        in_specs=[
            pl.BlockSpec((BLK, D), lambda i: (i, 0)),
            pl.BlockSpec((K, D), lambda i: (0, 0)),
            pl.BlockSpec((1, K), lambda i: (0, 0)),
        ],
        out_specs=pl.BlockSpec((1, 1), lambda i: (0, 0)),
        out_shape=jax.ShapeDtypeStruct((1, 1), jnp.float32),
        scratch_shapes=[
            pltpu.VMEM((K, D), jnp.float32),
            pltpu.VMEM((1, K), jnp.float32),
            pltpu.SMEM((1, 1), jnp.float32),
        ],
    )(z, centers, weights.reshape(1, K))
    return out[0, 0]
